# baseline (speedup 1.0000x reference)
import jax
import jax.numpy as jnp
from jax.experimental import pallas as pl

N_RAYS_K = 8192
T_THRESHOLD_K = 1e-4


def kernel(sigmas, rgbs, deltas, ts, segment_ids):
    alpha = 1.0 - jnp.exp(-sigmas * deltas)
    log1m = jnp.log(jnp.clip(1.0 - alpha, 1e-10, 1.0))
    csum = jnp.cumsum(log1m)
    excl = csum - log1m
    base = jax.ops.segment_max(excl, segment_ids, num_segments=N_RAYS_K)
    T = jnp.exp(excl - base[segment_ids])
    alive = T > T_THRESHOLD_K
    ws = jnp.where(alive, T * alpha, 0.0)
    opacity = jax.ops.segment_sum(ws, segment_ids, num_segments=N_RAYS_K)
    depth = jax.ops.segment_sum(ws * ts, segment_ids, num_segments=N_RAYS_K)
    rgb = jax.ops.segment_sum(ws[:, None] * rgbs, segment_ids, num_segments=N_RAYS_K)
    return opacity, depth, rgb, ws


# speedup vs baseline: 1.0947x; 1.0947x over previous
import functools

import jax
import jax.numpy as jnp
from jax.experimental import pallas as pl
from jax.experimental.pallas import tpu as pltpu

TOTAL_N = 524288
NRAYS = 8192
ROWS = 4096
BLK_R = 256
NSTEPS = ROWS // BLK_R
LOG_EPS = -23.025850929940457
T_THRESH = 1e-4

_DOT = functools.partial(jnp.dot, preferred_element_type=jnp.float32,
                         precision=jax.lax.Precision.HIGHEST)


def _tri_incl(k):
    a = jax.lax.broadcasted_iota(jnp.int32, (k, k), 0)
    b = jax.lax.broadcasted_iota(jnp.int32, (k, k), 1)
    return (a <= b).astype(jnp.float32)


def _tri_strict(k):
    a = jax.lax.broadcasted_iota(jnp.int32, (k, k), 0)
    b = jax.lax.broadcasted_iota(jnp.int32, (k, k), 1)
    return (a > b).astype(jnp.float32)


def _cumsum_block(A, carry):
    R = A.shape[0]
    rowinc = _DOT(A, _tri_incl(128))
    rs = rowinc[:, 127:128]
    off = _DOT(_tri_strict(R), rs)
    excl = carry + off + (rowinc - A)
    return excl, jnp.sum(rs)


def _excl_kernel(sig_ref, dlt_ref, out_ref, carry_ref):
    i = pl.program_id(0)

    @pl.when(i == 0)
    def _init():
        carry_ref[0] = 0.0

    A = jnp.maximum(-(sig_ref[...] * dlt_ref[...]), LOG_EPS)
    carry = carry_ref[0]
    excl, tot = _cumsum_block(A, carry)
    out_ref[...] = excl
    carry_ref[0] = carry + tot


def _excl_cumsum(sigmas, deltas):
    sig2 = sigmas.reshape(ROWS, 128)
    dlt2 = deltas.reshape(ROWS, 128)
    spec = pl.BlockSpec((BLK_R, 128), lambda i: (i, 0))
    return pl.pallas_call(
        _excl_kernel,
        grid=(NSTEPS,),
        in_specs=[spec, spec],
        out_specs=spec,
        out_shape=jax.ShapeDtypeStruct((ROWS, 128), jnp.float32),
        scratch_shapes=[pltpu.SMEM((1,), jnp.float32)],
    )(sig2, dlt2)


def _pass2_kernel(sig_ref, dlt_ref, ts_ref, r_ref, g_ref, b_ref,
                  excl_ref, base_ref, ws_ref, csum_ref, carry_ref):
    i = pl.program_id(0)

    @pl.when(i == 0)
    def _init():
        for c in range(5):
            carry_ref[c] = 0.0

    sd = sig_ref[...] * dlt_ref[...]
    alpha = 1.0 - jnp.exp(-sd)
    T = jnp.exp(excl_ref[...] - base_ref[...])
    ws = jnp.where(T > T_THRESH, T * alpha, 0.0)
    ws_ref[...] = ws
    chans = (ws, ws * ts_ref[...], ws * r_ref[...], ws * g_ref[...],
             ws * b_ref[...])
    U = _tri_incl(128)
    Ls = _tri_strict(BLK_R)
    for c, A in enumerate(chans):
        rowinc = _DOT(A, U)
        rs = rowinc[:, 127:128]
        off = _DOT(Ls, rs)
        csum_ref[c] = carry_ref[c] + off + rowinc
        carry_ref[c] = carry_ref[c] + jnp.sum(rs)


def _pass2(sigmas, deltas, ts, rc, gc, bc, excl, base_sample):
    args = [x.reshape(ROWS, 128) for x in
            (sigmas, deltas, ts, rc, gc, bc, excl, base_sample)]
    spec = pl.BlockSpec((BLK_R, 128), lambda i: (i, 0))
    cspec = pl.BlockSpec((5, BLK_R, 128), lambda i: (0, i, 0))
    ws2, csum = pl.pallas_call(
        _pass2_kernel,
        grid=(NSTEPS,),
        in_specs=[spec] * 8,
        out_specs=[spec, cspec],
        out_shape=[jax.ShapeDtypeStruct((ROWS, 128), jnp.float32),
                   jax.ShapeDtypeStruct((5, ROWS, 128), jnp.float32)],
        scratch_shapes=[pltpu.SMEM((5,), jnp.float32)],
    )(*args)
    return ws2, csum


def kernel(sigmas, rgbs, deltas, ts, segment_ids):
    excl = _excl_cumsum(sigmas, deltas)
    excl_flat = excl.reshape(TOTAL_N)

    rays = jnp.arange(NRAYS, dtype=jnp.int32)
    start = jnp.searchsorted(segment_ids, rays, side="left").astype(jnp.int32)
    base_ray = excl_flat[jnp.minimum(start, TOTAL_N - 1)]
    base_sample = base_ray[segment_ids]

    rc, gc, bc = rgbs[:, 0], rgbs[:, 1], rgbs[:, 2]
    ws2, csum = _pass2(sigmas, deltas, ts, rc, gc, bc, excl_flat, base_sample)
    ws = ws2.reshape(TOTAL_N)
    csum_flat = csum.reshape(5, TOTAL_N)

    end = jnp.concatenate([start[1:], jnp.array([TOTAL_N], jnp.int32)])
    pe = jnp.maximum(end - 1, 0)
    ps = jnp.maximum(start - 1, 0)
    ve = csum_flat[:, pe]
    vs = jnp.where(start[None, :] > 0, csum_flat[:, ps], 0.0)
    outs = jnp.where(end[None, :] > start[None, :], ve - vs, 0.0)
    opacity = outs[0]
    depth = outs[1]
    rgb = outs[2:5].T
    return opacity, depth, rgb, ws


# speedup vs baseline: 32.0011x; 29.2339x over previous
import functools

import jax
import jax.numpy as jnp
from jax import lax
from jax.experimental import pallas as pl
from jax.experimental.pallas import tpu as pltpu
from jax.experimental.pallas import tpu_sc as plsc

TOTAL_N = 524288
NRAYS = 8192
ROWS = 4096
BLK_R = 256
NSTEPS = ROWS // BLK_R
LOG_EPS = -23.025850929940457
T_THRESH = 1e-4

NTILES = 32
RAYS_PT = NRAYS // NTILES
SAMP_PT = TOTAL_N // NTILES
START_PAD = NRAYS + 64

_DOT = functools.partial(jnp.dot, preferred_element_type=jnp.float32,
                         precision=jax.lax.Precision.HIGHEST)


def _tri_incl(k):
    a = jax.lax.broadcasted_iota(jnp.int32, (k, k), 0)
    b = jax.lax.broadcasted_iota(jnp.int32, (k, k), 1)
    return (a <= b).astype(jnp.float32)


def _tri_strict(k):
    a = jax.lax.broadcasted_iota(jnp.int32, (k, k), 0)
    b = jax.lax.broadcasted_iota(jnp.int32, (k, k), 1)
    return (a > b).astype(jnp.float32)


def _excl_kernel(sig_ref, dlt_ref, out_ref, carry_ref):
    i = pl.program_id(0)

    @pl.when(i == 0)
    def _init():
        carry_ref[0] = 0.0

    A = jnp.maximum(-(sig_ref[...] * dlt_ref[...]), LOG_EPS)
    rowinc = _DOT(A, _tri_incl(128))
    rs = rowinc[:, 127:128]
    off = _DOT(_tri_strict(BLK_R), rs)
    carry = carry_ref[0]
    out_ref[...] = carry + off + (rowinc - A)
    carry_ref[0] = carry + jnp.sum(rs)


def _excl_cumsum(sigmas, deltas):
    spec = pl.BlockSpec((BLK_R, 128), lambda i: (i, 0))
    return pl.pallas_call(
        _excl_kernel,
        grid=(NSTEPS,),
        in_specs=[spec, spec],
        out_specs=spec,
        out_shape=jax.ShapeDtypeStruct((ROWS, 128), jnp.float32),
        scratch_shapes=[pltpu.SMEM((1,), jnp.float32)],
    )(sigmas.reshape(ROWS, 128), deltas.reshape(ROWS, 128))


def _pass2_kernel(sig_ref, dlt_ref, ts_ref, r_ref, g_ref, b_ref,
                  excl_ref, base_ref, ws_ref, csum_ref, carry_ref):
    i = pl.program_id(0)

    @pl.when(i == 0)
    def _init():
        for c in range(5):
            carry_ref[c] = 0.0

    sd = sig_ref[...] * dlt_ref[...]
    alpha = 1.0 - jnp.exp(-sd)
    T = jnp.exp(excl_ref[...] - base_ref[...])
    ws = jnp.where(T > T_THRESH, T * alpha, 0.0)
    ws_ref[...] = ws
    chans = (ws, ws * ts_ref[...], ws * r_ref[...], ws * g_ref[...],
             ws * b_ref[...])
    U = _tri_incl(128)
    Ls = _tri_strict(BLK_R)
    for c, A in enumerate(chans):
        rowinc = _DOT(A, U)
        rs = rowinc[:, 127:128]
        off = _DOT(Ls, rs)
        csum_ref[c] = carry_ref[c] + off + rowinc
        carry_ref[c] = carry_ref[c] + jnp.sum(rs)


def _pass2(sigmas, deltas, ts, rc, gc, bc, excl_flat, base_sample):
    args = [x.reshape(ROWS, 128) for x in
            (sigmas, deltas, ts, rc, gc, bc, excl_flat, base_sample)]
    spec = pl.BlockSpec((BLK_R, 128), lambda i: (i, 0))
    cspec = pl.BlockSpec((5, BLK_R, 128), lambda i: (0, i, 0))
    return pl.pallas_call(
        _pass2_kernel,
        grid=(NSTEPS,),
        in_specs=[spec] * 8,
        out_specs=[spec, cspec],
        out_shape=[jax.ShapeDtypeStruct((ROWS, 128), jnp.float32),
                   jax.ShapeDtypeStruct((5, ROWS, 128), jnp.float32)],
        scratch_shapes=[pltpu.SMEM((5,), jnp.float32)],
    )(*args)


def _sc_mesh():
    return plsc.VectorSubcoreMesh(core_axis_name="c", subcore_axis_name="s")


_SC_PARAMS = pltpu.CompilerParams(needs_layout_passes=False)


def _wid():
    return lax.axis_index("c") * 16 + lax.axis_index("s")


_IOTA16 = functools.partial(lax.iota, jnp.int32, 16)


def _sc_start_base(segment_ids, excl_flat):
    @functools.partial(
        pl.kernel,
        out_type=[jax.ShapeDtypeStruct((START_PAD,), jnp.int32),
                  jax.ShapeDtypeStruct((NRAYS,), jnp.float32)],
        mesh=_sc_mesh(),
        compiler_params=_SC_PARAMS,
        scratch_types=[
            pltpu.VMEM((2, 128), jnp.int32),
            pltpu.VMEM((2, 128), jnp.int32),
            pltpu.VMEM((2, 128), jnp.int32),
            pltpu.VMEM((2, 128), jnp.int32),
            pltpu.VMEM((2, 128), jnp.float32),
            pltpu.VMEM((64,), jnp.int32),
            pltpu.SemaphoreType.DMA,
            pltpu.SemaphoreType.DMA,
        ],
    )
    def k(seg_hbm, excl_hbm, start_hbm, base_hbm,
          lo_r, hi_r, mid_r, sm_r, bv_r, pad_r, sem0, sem1):
        wid = _wid()
        rbase = wid * RAYS_PT
        zeros = jnp.zeros((16,), jnp.int32)
        total = jnp.full((16,), TOTAL_N, jnp.int32)
        for j in range(2):
            row_lo = lo_r.at[j]
            row_hi = hi_r.at[j]
            for kk in range(8):
                row_lo[pl.ds(kk * 16, 16)] = zeros
                row_hi[pl.ds(kk * 16, 16)] = total

        def round_body(_, carry):
            for j in range(2):
                for kk in range(8):
                    lo = lo_r.at[j][pl.ds(kk * 16, 16)]
                    hi = hi_r.at[j][pl.ds(kk * 16, 16)]
                    mid = lax.shift_right_logical(lo + hi, 1)
                    mid_r.at[j][pl.ds(kk * 16, 16)] = jnp.minimum(
                        mid, TOTAL_N - 1)
            d0 = pltpu.async_copy(seg_hbm.at[mid_r.at[0]], sm_r.at[0], sem0)
            d1 = pltpu.async_copy(seg_hbm.at[mid_r.at[1]], sm_r.at[1], sem1)
            d0.wait()
            d1.wait()
            for j in range(2):
                for kk in range(8):
                    lo = lo_r.at[j][pl.ds(kk * 16, 16)]
                    hi = hi_r.at[j][pl.ds(kk * 16, 16)]
                    mid = lax.shift_right_logical(lo + hi, 1)
                    sm = sm_r.at[j][pl.ds(kk * 16, 16)]
                    r = rbase + j * 128 + kk * 16 + _IOTA16()
                    active = lo < hi
                    pred = sm < r
                    lo_r.at[j][pl.ds(kk * 16, 16)] = jnp.where(
                        active, jnp.where(pred, mid + 1, lo), lo)
                    hi_r.at[j][pl.ds(kk * 16, 16)] = jnp.where(
                        active, jnp.where(pred, hi, mid), hi)
            return carry

        lax.fori_loop(0, 20, round_body, 0)

        for j in range(2):
            pltpu.sync_copy(lo_r.at[j], start_hbm.at[pl.ds(
                rbase + 128 * j, 128)])
            for kk in range(8):
                lo = lo_r.at[j][pl.ds(kk * 16, 16)]
                mid_r.at[j][pl.ds(kk * 16, 16)] = jnp.minimum(lo, TOTAL_N - 1)
        d0 = pltpu.async_copy(excl_hbm.at[mid_r.at[0]], bv_r.at[0], sem0)
        d1 = pltpu.async_copy(excl_hbm.at[mid_r.at[1]], bv_r.at[1], sem1)
        d0.wait()
        d1.wait()
        for j in range(2):
            pltpu.sync_copy(bv_r.at[j], base_hbm.at[pl.ds(
                rbase + 128 * j, 128)])

        @pl.when(wid == NTILES - 1)
        def _pad():
            for kk in range(4):
                pad_r[pl.ds(kk * 16, 16)] = total
            pltpu.sync_copy(pad_r, start_hbm.at[pl.ds(NRAYS, 64)])

    return k(segment_ids, excl_flat)


def _sc_expand(segment_ids, base_ray):
    @functools.partial(
        pl.kernel,
        out_type=jax.ShapeDtypeStruct((TOTAL_N,), jnp.float32),
        mesh=_sc_mesh(),
        compiler_params=_SC_PARAMS,
        scratch_types=[
            pltpu.VMEM((NRAYS,), jnp.float32),
            pltpu.VMEM((SAMP_PT,), jnp.int32),
            pltpu.VMEM((SAMP_PT,), jnp.float32),
        ],
    )
    def k(seg_hbm, base_hbm, out_hbm, tab_r, seg_r, out_r):
        wid = _wid()
        sbase = wid * SAMP_PT
        pltpu.sync_copy(base_hbm, tab_r)
        pltpu.sync_copy(seg_hbm.at[pl.ds(sbase, SAMP_PT)], seg_r)

        def body(i, carry):
            off = i * 16
            s = seg_r[pl.ds(off, 16)]
            out_r[pl.ds(off, 16)] = plsc.load_gather(tab_r, [s])
            return carry

        lax.fori_loop(0, SAMP_PT // 16, body, 0, unroll=8)
        pltpu.sync_copy(out_r, out_hbm.at[pl.ds(sbase, SAMP_PT)])

    return k(segment_ids, base_ray)


def _sc_finalize(csum_flat, start_pad):
    @functools.partial(
        pl.kernel,
        out_type=jax.ShapeDtypeStruct((5 * NRAYS,), jnp.float32),
        mesh=_sc_mesh(),
        compiler_params=_SC_PARAMS,
        scratch_types=[
            pltpu.VMEM((264,), jnp.int32),
            pltpu.VMEM((20, 128), jnp.int32),
            pltpu.VMEM((20, 128), jnp.float32),
            pltpu.VMEM((10, 128), jnp.float32),
            pltpu.SemaphoreType.DMA,
        ],
    )
    def k(csum_hbm, start_hbm, out_hbm, st_r, idx_r, val_r, ob_r, sem):
        wid = _wid()
        rbase = wid * RAYS_PT
        pltpu.sync_copy(start_hbm.at[pl.ds(rbase, 264)], st_r)
        for j in range(2):
            for kk in range(8):
                iv = j * 128 + kk * 16 + _IOTA16()
                s = plsc.load_gather(st_r, [iv])
                e = plsc.load_gather(st_r, [iv + 1])
                ps = jnp.maximum(s - 1, 0)
                pe = jnp.maximum(e - 1, 0)
                for c in range(5):
                    idx_r.at[(c * 2) * 2 + j][pl.ds(kk * 16, 16)] = (
                        ps + c * TOTAL_N)
                    idx_r.at[(c * 2 + 1) * 2 + j][pl.ds(kk * 16, 16)] = (
                        pe + c * TOTAL_N)
        copies = [pltpu.async_copy(csum_hbm.at[idx_r.at[row]],
                                   val_r.at[row], sem)
                  for row in range(20)]
        for cp in copies:
            cp.wait()
        zero = jnp.zeros((16,), jnp.float32)
        for j in range(2):
            for kk in range(8):
                iv = j * 128 + kk * 16 + _IOTA16()
                s = plsc.load_gather(st_r, [iv])
                e = plsc.load_gather(st_r, [iv + 1])
                nonempty = e > s
                haveprev = s > 0
                for c in range(5):
                    vs = val_r.at[(c * 2) * 2 + j][pl.ds(kk * 16, 16)]
                    ve = val_r.at[(c * 2 + 1) * 2 + j][pl.ds(kk * 16, 16)]
                    res = jnp.where(
                        nonempty,
                        ve - jnp.where(haveprev, vs, zero), zero)
                    ob_r.at[c * 2 + j][pl.ds(kk * 16, 16)] = res
        for c in range(5):
            for j in range(2):
                pltpu.sync_copy(ob_r.at[c * 2 + j],
                                out_hbm.at[pl.ds(c * NRAYS + rbase + 128 * j,
                                                 128)])

    return k(csum_flat, start_pad)


def kernel(sigmas, rgbs, deltas, ts, segment_ids):
    excl_flat = _excl_cumsum(sigmas, deltas).reshape(TOTAL_N)
    start_pad, base_ray = _sc_start_base(segment_ids, excl_flat)
    base_sample = _sc_expand(segment_ids, base_ray)
    rc, gc, bc = rgbs[:, 0], rgbs[:, 1], rgbs[:, 2]
    ws2, csum = _pass2(sigmas, deltas, ts, rc, gc, bc, excl_flat, base_sample)
    outs = _sc_finalize(csum.reshape(5 * TOTAL_N), start_pad).reshape(5, NRAYS)
    opacity = outs[0]
    depth = outs[1]
    rgb = outs[2:5].T
    return opacity, depth, rgb, ws2.reshape(TOTAL_N)


# speedup vs baseline: 32.3472x; 1.0108x over previous
import functools

import jax
import jax.numpy as jnp
from jax import lax
from jax.experimental import pallas as pl
from jax.experimental.pallas import tpu as pltpu
from jax.experimental.pallas import tpu_sc as plsc

TOTAL_N = 524288
NRAYS = 8192
ROWS = 4096
BLK_R = 256
NSTEPS = ROWS // BLK_R
LOG_EPS = -23.025850929940457
T_THRESH = 1e-4

NTILES = 32
RAYS_PT = NRAYS // NTILES
SAMP_PT = TOTAL_N // NTILES
START_PAD = NRAYS + 64
WIN = 128
NSUB = TOTAL_N // WIN

_DOT = functools.partial(jnp.dot, preferred_element_type=jnp.float32,
                         precision=jax.lax.Precision.HIGHEST)


def _tri_incl(k):
    a = jax.lax.broadcasted_iota(jnp.int32, (k, k), 0)
    b = jax.lax.broadcasted_iota(jnp.int32, (k, k), 1)
    return (a <= b).astype(jnp.float32)


def _tri_strict(k):
    a = jax.lax.broadcasted_iota(jnp.int32, (k, k), 0)
    b = jax.lax.broadcasted_iota(jnp.int32, (k, k), 1)
    return (a > b).astype(jnp.float32)


def _scan_lanes(x):
    n = x.shape[1]
    k = 1
    while k < n:
        shifted = jnp.concatenate(
            [jnp.zeros((x.shape[0], k), x.dtype), x[:, :n - k]], axis=1)
        x = x + shifted
        k *= 2
    return x


def _scan_rows(rs):
    n = rs.shape[0]
    k = 1
    while k < n:
        shifted = jnp.concatenate(
            [jnp.zeros((k, 1), rs.dtype), rs[:n - k]], axis=0)
        rs = rs + shifted
        k *= 2
    return rs


def _block_cumsum(A, carry):
    rowinc = _scan_lanes(A)
    rs = rowinc[:, 127:128]
    offinc = _scan_rows(rs)
    incl = carry + (offinc - rs) + rowinc
    return incl, jnp.sum(rs)


def _excl_kernel(sig_ref, dlt_ref, out_ref, carry_ref):
    i = pl.program_id(0)

    @pl.when(i == 0)
    def _init():
        carry_ref[0] = 0.0

    A = jnp.maximum(-(sig_ref[...] * dlt_ref[...]), LOG_EPS)
    carry = carry_ref[0]
    incl, tot = _block_cumsum(A, carry)
    out_ref[...] = incl - A
    carry_ref[0] = carry + tot


def _excl_cumsum(sigmas, deltas):
    spec = pl.BlockSpec((BLK_R, 128), lambda i: (i, 0))
    return pl.pallas_call(
        _excl_kernel,
        grid=(NSTEPS,),
        in_specs=[spec, spec],
        out_specs=spec,
        out_shape=jax.ShapeDtypeStruct((ROWS, 128), jnp.float32),
        scratch_shapes=[pltpu.SMEM((1,), jnp.float32)],
    )(sigmas.reshape(ROWS, 128), deltas.reshape(ROWS, 128))


def _pass2_kernel(sig_ref, dlt_ref, ts_ref, r_ref, g_ref, b_ref,
                  excl_ref, base_ref, ws_ref, csum_ref, carry_ref):
    i = pl.program_id(0)

    @pl.when(i == 0)
    def _init():
        for c in range(5):
            carry_ref[c] = 0.0

    sd = sig_ref[...] * dlt_ref[...]
    alpha = 1.0 - jnp.exp(-sd)
    T = jnp.exp(excl_ref[...] - base_ref[...])
    ws = jnp.where(T > T_THRESH, T * alpha, 0.0)
    ws_ref[...] = ws
    chans = (ws, ws * ts_ref[...], ws * r_ref[...], ws * g_ref[...],
             ws * b_ref[...])
    for c, A in enumerate(chans):
        incl, tot = _block_cumsum(A, carry_ref[c])
        csum_ref[c] = incl
        carry_ref[c] = carry_ref[c] + tot


def _pass2(sigmas, deltas, ts, rc, gc, bc, excl_flat, base_sample):
    args = [x.reshape(ROWS, 128) for x in
            (sigmas, deltas, ts, rc, gc, bc, excl_flat, base_sample)]
    spec = pl.BlockSpec((BLK_R, 128), lambda i: (i, 0))
    cspec = pl.BlockSpec((5, BLK_R, 128), lambda i: (0, i, 0))
    return pl.pallas_call(
        _pass2_kernel,
        grid=(NSTEPS,),
        in_specs=[spec] * 8,
        out_specs=[spec, cspec],
        out_shape=[jax.ShapeDtypeStruct((ROWS, 128), jnp.float32),
                   jax.ShapeDtypeStruct((5, ROWS, 128), jnp.float32)],
        scratch_shapes=[pltpu.SMEM((5,), jnp.float32)],
    )(*args)


def _sc_mesh():
    return plsc.VectorSubcoreMesh(core_axis_name="c", subcore_axis_name="s")


_SC_PARAMS = pltpu.CompilerParams(needs_layout_passes=False)


def _wid():
    return lax.axis_index("c") * 16 + lax.axis_index("s")


_IOTA16 = functools.partial(lax.iota, jnp.int32, 16)


def _sc_start_base(seg2d, seg_sub, excl_flat):
    @functools.partial(
        pl.kernel,
        out_type=[jax.ShapeDtypeStruct((START_PAD,), jnp.int32),
                  jax.ShapeDtypeStruct((NRAYS,), jnp.float32)],
        mesh=_sc_mesh(),
        compiler_params=_SC_PARAMS,
        scratch_types=[
            pltpu.VMEM((NSUB,), jnp.int32),
            pltpu.VMEM((2, 128), jnp.int32),
            pltpu.VMEM((256, 128), jnp.int32),
            pltpu.VMEM((2, 128), jnp.float32),
            pltpu.VMEM((64,), jnp.int32),
            pltpu.SemaphoreType.DMA,
            pltpu.SemaphoreType.DMA,
        ],
    )
    def k(seg2d_hbm, sub_hbm, excl_hbm, start_hbm, base_hbm,
          sub_r, wi_r, rows_r, bv_r, pad_r, sem0, sem1):
        wid = _wid()
        rbase = wid * RAYS_PT
        total = jnp.full((16,), TOTAL_N, jnp.int32)
        pltpu.sync_copy(sub_hbm, sub_r)

        cs_vecs = []
        for j in range(2):
            for kk in range(8):
                r = rbase + j * 128 + kk * 16 + _IOTA16()
                lo = jnp.zeros((16,), jnp.int32)
                hi = jnp.full((16,), NSUB, jnp.int32)
                for _ in range(13):
                    mid = lax.shift_right_logical(lo + hi, 1)
                    sv = plsc.load_gather(
                        sub_r, [jnp.minimum(mid, NSUB - 1)])
                    active = lo < hi
                    pred = sv < r
                    lo = jnp.where(active, jnp.where(pred, mid + 1, lo), lo)
                    hi = jnp.where(active, jnp.where(pred, hi, mid), hi)
                cs_vecs.append(lo)
                wi_r.at[j][pl.ds(kk * 16, 16)] = jnp.maximum(lo - 1, 0)

        d0 = pltpu.async_copy(seg2d_hbm.at[wi_r.at[0]],
                              rows_r.at[pl.ds(0, 128)], sem0)
        d1 = pltpu.async_copy(seg2d_hbm.at[wi_r.at[1]],
                              rows_r.at[pl.ds(128, 128)], sem1)
        d0.wait()
        d1.wait()

        for j in range(2):
            for kk in range(8):
                v = j * 8 + kk
                cs = cs_vecs[v]
                r = rbase + j * 128 + kk * 16 + _IOTA16()
                lr = j * 128 + kk * 16 + _IOTA16()
                lo = jnp.ones((16,), jnp.int32)
                hi = jnp.full((16,), WIN, jnp.int32)
                for _ in range(7):
                    mid = lax.shift_right_logical(lo + hi, 1)
                    sv = plsc.load_gather(rows_r, [lr, mid])
                    active = lo < hi
                    pred = sv < r
                    lo = jnp.where(active, jnp.where(pred, mid + 1, lo), lo)
                    hi = jnp.where(active, jnp.where(pred, hi, mid), hi)
                w = jnp.maximum(cs - 1, 0)
                st = jnp.where(cs > 0, w * WIN + lo, 0)
                wi_r.at[j][pl.ds(kk * 16, 16)] = st

        for j in range(2):
            pltpu.sync_copy(wi_r.at[j], start_hbm.at[pl.ds(
                rbase + 128 * j, 128)])
        for j in range(2):
            for kk in range(8):
                st = wi_r.at[j][pl.ds(kk * 16, 16)]
                wi_r.at[j][pl.ds(kk * 16, 16)] = jnp.minimum(st, TOTAL_N - 1)
        d0 = pltpu.async_copy(excl_hbm.at[wi_r.at[0]], bv_r.at[0], sem0)
        d1 = pltpu.async_copy(excl_hbm.at[wi_r.at[1]], bv_r.at[1], sem1)
        d0.wait()
        d1.wait()
        for j in range(2):
            pltpu.sync_copy(bv_r.at[j], base_hbm.at[pl.ds(
                rbase + 128 * j, 128)])

        @pl.when(wid == NTILES - 1)
        def _pad():
            for kk in range(4):
                pad_r[pl.ds(kk * 16, 16)] = total
            pltpu.sync_copy(pad_r, start_hbm.at[pl.ds(NRAYS, 64)])

    return k(seg2d, seg_sub, excl_flat)


def _sc_expand(segment_ids, base_ray):
    @functools.partial(
        pl.kernel,
        out_type=jax.ShapeDtypeStruct((TOTAL_N,), jnp.float32),
        mesh=_sc_mesh(),
        compiler_params=_SC_PARAMS,
        scratch_types=[
            pltpu.VMEM((NRAYS,), jnp.float32),
            pltpu.VMEM((SAMP_PT,), jnp.int32),
            pltpu.VMEM((SAMP_PT,), jnp.float32),
        ],
    )
    def k(seg_hbm, base_hbm, out_hbm, tab_r, seg_r, out_r):
        wid = _wid()
        sbase = wid * SAMP_PT
        pltpu.sync_copy(base_hbm, tab_r)
        pltpu.sync_copy(seg_hbm.at[pl.ds(sbase, SAMP_PT)], seg_r)

        def body(i, carry):
            off = i * 16
            s = seg_r[pl.ds(off, 16)]
            out_r[pl.ds(off, 16)] = plsc.load_gather(tab_r, [s])
            return carry

        lax.fori_loop(0, SAMP_PT // 16, body, 0, unroll=8)
        pltpu.sync_copy(out_r, out_hbm.at[pl.ds(sbase, SAMP_PT)])

    return k(segment_ids, base_ray)


def _sc_finalize(csum_flat, start_pad):
    @functools.partial(
        pl.kernel,
        out_type=jax.ShapeDtypeStruct((5 * NRAYS,), jnp.float32),
        mesh=_sc_mesh(),
        compiler_params=_SC_PARAMS,
        scratch_types=[
            pltpu.VMEM((264,), jnp.int32),
            pltpu.VMEM((20, 128), jnp.int32),
            pltpu.VMEM((20, 128), jnp.float32),
            pltpu.VMEM((10, 128), jnp.float32),
            pltpu.SemaphoreType.DMA,
        ],
    )
    def k(csum_hbm, start_hbm, out_hbm, st_r, idx_r, val_r, ob_r, sem):
        wid = _wid()
        rbase = wid * RAYS_PT
        pltpu.sync_copy(start_hbm.at[pl.ds(rbase, 264)], st_r)
        for j in range(2):
            for kk in range(8):
                iv = j * 128 + kk * 16 + _IOTA16()
                s = plsc.load_gather(st_r, [iv])
                e = plsc.load_gather(st_r, [iv + 1])
                ps = jnp.maximum(s - 1, 0)
                pe = jnp.maximum(e - 1, 0)
                for c in range(5):
                    idx_r.at[(c * 2) * 2 + j][pl.ds(kk * 16, 16)] = (
                        ps + c * TOTAL_N)
                    idx_r.at[(c * 2 + 1) * 2 + j][pl.ds(kk * 16, 16)] = (
                        pe + c * TOTAL_N)
        copies = [pltpu.async_copy(csum_hbm.at[idx_r.at[row]],
                                   val_r.at[row], sem)
                  for row in range(20)]
        for cp in copies:
            cp.wait()
        zero = jnp.zeros((16,), jnp.float32)
        for j in range(2):
            for kk in range(8):
                iv = j * 128 + kk * 16 + _IOTA16()
                s = plsc.load_gather(st_r, [iv])
                e = plsc.load_gather(st_r, [iv + 1])
                nonempty = e > s
                haveprev = s > 0
                for c in range(5):
                    vs = val_r.at[(c * 2) * 2 + j][pl.ds(kk * 16, 16)]
                    ve = val_r.at[(c * 2 + 1) * 2 + j][pl.ds(kk * 16, 16)]
                    res = jnp.where(
                        nonempty,
                        ve - jnp.where(haveprev, vs, zero), zero)
                    ob_r.at[c * 2 + j][pl.ds(kk * 16, 16)] = res
        for c in range(5):
            for j in range(2):
                pltpu.sync_copy(ob_r.at[c * 2 + j],
                                out_hbm.at[pl.ds(c * NRAYS + rbase + 128 * j,
                                                 128)])

    return k(csum_flat, start_pad)


def kernel(sigmas, rgbs, deltas, ts, segment_ids):
    excl_flat = _excl_cumsum(sigmas, deltas).reshape(TOTAL_N)
    seg2d = segment_ids.reshape(NSUB, WIN)
    seg_sub = seg2d[:, 0]
    start_pad, base_ray = _sc_start_base(seg2d, seg_sub, excl_flat)
    base_sample = _sc_expand(segment_ids, base_ray)
    rc, gc, bc = rgbs[:, 0], rgbs[:, 1], rgbs[:, 2]
    ws2, csum = _pass2(sigmas, deltas, ts, rc, gc, bc, excl_flat, base_sample)
    outs = _sc_finalize(csum.reshape(5 * TOTAL_N), start_pad).reshape(5, NRAYS)
    opacity = outs[0]
    depth = outs[1]
    rgb = outs[2:5].T
    return opacity, depth, rgb, ws2.reshape(TOTAL_N)


# speedup vs baseline: 69.5309x; 2.1495x over previous
import functools

import jax
import jax.numpy as jnp
from jax import lax
from jax.experimental import pallas as pl
from jax.experimental.pallas import tpu as pltpu
from jax.experimental.pallas import tpu_sc as plsc

TOTAL_N = 524288
NRAYS = 8192
ROWS = 4096
BLK_R = 256
NSTEPS = ROWS // BLK_R
LOG_EPS = -23.025850929940457
T_THRESH = 1e-4

NTILES = 32
RAYS_PT = NRAYS // NTILES
SAMP_PT = TOTAL_N // NTILES
START_PAD = NRAYS + 64
WIN = 128
NSUB = TOTAL_N // WIN

_DOT = functools.partial(jnp.dot, preferred_element_type=jnp.float32,
                         precision=jax.lax.Precision.HIGHEST)


def _tri_incl(k):
    a = jax.lax.broadcasted_iota(jnp.int32, (k, k), 0)
    b = jax.lax.broadcasted_iota(jnp.int32, (k, k), 1)
    return (a <= b).astype(jnp.float32)


def _tri_strict(k):
    a = jax.lax.broadcasted_iota(jnp.int32, (k, k), 0)
    b = jax.lax.broadcasted_iota(jnp.int32, (k, k), 1)
    return (a > b).astype(jnp.float32)


def _scan_lanes(x):
    lane = jax.lax.broadcasted_iota(jnp.int32, x.shape, 1)
    k = 1
    while k < x.shape[1]:
        x = x + jnp.where(lane >= k, pltpu.roll(x, k, 1), 0.0)
        k *= 2
    return x


def _scan_sub(x):
    row = jax.lax.broadcasted_iota(jnp.int32, x.shape, 0)
    k = 1
    while k < x.shape[0]:
        x = x + jnp.where(row >= k, pltpu.roll(x, k, 0), 0.0)
        k *= 2
    return x


def _block_cumsum(A, carry):
    S = _scan_lanes(A)
    G = _scan_sub(S)
    off = G[:, 127:128] - S[:, 127:128]
    incl = carry + off + S
    return incl, jnp.sum(A)


def _block_cumsum_multi(chans, carries):
    U = _tri_incl(128)
    Ss = [jnp.dot(A, U, preferred_element_type=jnp.float32) for A in chans]
    rt = jnp.concatenate([S[:, 127:128] for S in Ss], axis=1)
    G = _scan_sub(rt)
    off = G - rt
    incls = [carries[c] + off[:, c:c + 1] + Ss[c]
             for c in range(len(chans))]
    tots = [jnp.sum(A) for A in chans]
    return incls, tots


def _excl_kernel(sig_ref, dlt_ref, out_ref, carry_ref):
    i = pl.program_id(0)

    @pl.when(i == 0)
    def _init():
        carry_ref[0] = 0.0

    A = jnp.maximum(-(sig_ref[...] * dlt_ref[...]), LOG_EPS)
    carry = carry_ref[0]
    S = _DOT(A, _tri_incl(128))
    G = _scan_sub(S)
    off = G[:, 127:128] - S[:, 127:128]
    out_ref[...] = carry + off + (S - A)
    carry_ref[0] = carry + jnp.sum(A)


def _excl_cumsum(sigmas, deltas):
    spec = pl.BlockSpec((BLK_R, 128), lambda i: (i, 0))
    return pl.pallas_call(
        _excl_kernel,
        grid=(NSTEPS,),
        in_specs=[spec, spec],
        out_specs=spec,
        out_shape=jax.ShapeDtypeStruct((ROWS, 128), jnp.float32),
        scratch_shapes=[pltpu.SMEM((1,), jnp.float32)],
    )(sigmas.reshape(ROWS, 128), deltas.reshape(ROWS, 128))


def _pass2_kernel(sig_ref, dlt_ref, ts_ref, r_ref, g_ref, b_ref,
                  excl_ref, base_ref, ws_ref, csum_ref, carry_ref):
    i = pl.program_id(0)

    @pl.when(i == 0)
    def _init():
        for c in range(5):
            carry_ref[c] = 0.0

    sd = sig_ref[...] * dlt_ref[...]
    alpha = 1.0 - jnp.exp(-sd)
    T = jnp.exp(excl_ref[...] - base_ref[...])
    ws = jnp.where(T > T_THRESH, T * alpha, 0.0)
    ws_ref[...] = ws
    chans = (ws, ws * ts_ref[...], ws * r_ref[...], ws * g_ref[...],
             ws * b_ref[...])
    incls, tots = _block_cumsum_multi(chans, [carry_ref[c] for c in range(5)])
    for c in range(5):
        csum_ref[c] = incls[c]
        carry_ref[c] = carry_ref[c] + tots[c]


def _pass2(sigmas, deltas, ts, rc, gc, bc, excl_flat, base_sample):
    args = [x.reshape(ROWS, 128) for x in
            (sigmas, deltas, ts, rc, gc, bc, excl_flat, base_sample)]
    spec = pl.BlockSpec((BLK_R, 128), lambda i: (i, 0))
    cspec = pl.BlockSpec((5, BLK_R, 128), lambda i: (0, i, 0))
    return pl.pallas_call(
        _pass2_kernel,
        grid=(NSTEPS,),
        in_specs=[spec] * 8,
        out_specs=[spec, cspec],
        out_shape=[jax.ShapeDtypeStruct((ROWS, 128), jnp.float32),
                   jax.ShapeDtypeStruct((5, ROWS, 128), jnp.float32)],
        scratch_shapes=[pltpu.SMEM((5,), jnp.float32)],
    )(*args)


def _sc_mesh():
    return plsc.VectorSubcoreMesh(core_axis_name="c", subcore_axis_name="s")


_SC_PARAMS = pltpu.CompilerParams(needs_layout_passes=False)


def _wid():
    return lax.axis_index("c") * 16 + lax.axis_index("s")


_IOTA16 = functools.partial(lax.iota, jnp.int32, 16)


def _sc_start_base(seg2d, seg_sub, excl_flat):
    @functools.partial(
        pl.kernel,
        out_type=[jax.ShapeDtypeStruct((START_PAD,), jnp.int32),
                  jax.ShapeDtypeStruct((NRAYS,), jnp.float32)],
        mesh=_sc_mesh(),
        compiler_params=_SC_PARAMS,
        scratch_types=[
            pltpu.VMEM((NSUB,), jnp.int32),
            pltpu.VMEM((2, 128), jnp.int32),
            pltpu.VMEM((256, 128), jnp.int32),
            pltpu.VMEM((2, 128), jnp.float32),
            pltpu.VMEM((64,), jnp.int32),
            pltpu.SemaphoreType.DMA,
            pltpu.SemaphoreType.DMA,
        ],
    )
    def k(seg2d_hbm, sub_hbm, excl_hbm, start_hbm, base_hbm,
          sub_r, wi_r, rows_r, bv_r, pad_r, sem0, sem1):
        wid = _wid()
        rbase = wid * RAYS_PT
        total = jnp.full((16,), TOTAL_N, jnp.int32)
        pltpu.sync_copy(sub_hbm, sub_r)

        cs_vecs = []
        for j in range(2):
            for kk in range(8):
                r = rbase + j * 128 + kk * 16 + _IOTA16()
                lo = jnp.zeros((16,), jnp.int32)
                hi = jnp.full((16,), NSUB, jnp.int32)
                for _ in range(13):
                    mid = lax.shift_right_logical(lo + hi, 1)
                    sv = plsc.load_gather(
                        sub_r, [jnp.minimum(mid, NSUB - 1)])
                    active = lo < hi
                    pred = sv < r
                    lo = jnp.where(active, jnp.where(pred, mid + 1, lo), lo)
                    hi = jnp.where(active, jnp.where(pred, hi, mid), hi)
                cs_vecs.append(lo)
                wi_r.at[j][pl.ds(kk * 16, 16)] = jnp.maximum(lo - 1, 0)

        d0 = pltpu.async_copy(seg2d_hbm.at[wi_r.at[0]],
                              rows_r.at[pl.ds(0, 128)], sem0)
        d1 = pltpu.async_copy(seg2d_hbm.at[wi_r.at[1]],
                              rows_r.at[pl.ds(128, 128)], sem1)
        d0.wait()
        d1.wait()

        for j in range(2):
            for kk in range(8):
                v = j * 8 + kk
                cs = cs_vecs[v]
                r = rbase + j * 128 + kk * 16 + _IOTA16()
                lr = j * 128 + kk * 16 + _IOTA16()
                lo = jnp.ones((16,), jnp.int32)
                hi = jnp.full((16,), WIN, jnp.int32)
                for _ in range(7):
                    mid = lax.shift_right_logical(lo + hi, 1)
                    sv = plsc.load_gather(rows_r, [lr, mid])
                    active = lo < hi
                    pred = sv < r
                    lo = jnp.where(active, jnp.where(pred, mid + 1, lo), lo)
                    hi = jnp.where(active, jnp.where(pred, hi, mid), hi)
                w = jnp.maximum(cs - 1, 0)
                st = jnp.where(cs > 0, w * WIN + lo, 0)
                wi_r.at[j][pl.ds(kk * 16, 16)] = st

        for j in range(2):
            pltpu.sync_copy(wi_r.at[j], start_hbm.at[pl.ds(
                rbase + 128 * j, 128)])
        for j in range(2):
            for kk in range(8):
                st = wi_r.at[j][pl.ds(kk * 16, 16)]
                wi_r.at[j][pl.ds(kk * 16, 16)] = jnp.minimum(st, TOTAL_N - 1)
        d0 = pltpu.async_copy(excl_hbm.at[wi_r.at[0]], bv_r.at[0], sem0)
        d1 = pltpu.async_copy(excl_hbm.at[wi_r.at[1]], bv_r.at[1], sem1)
        d0.wait()
        d1.wait()
        for j in range(2):
            pltpu.sync_copy(bv_r.at[j], base_hbm.at[pl.ds(
                rbase + 128 * j, 128)])

        @pl.when(wid == NTILES - 1)
        def _pad():
            for kk in range(4):
                pad_r[pl.ds(kk * 16, 16)] = total
            pltpu.sync_copy(pad_r, start_hbm.at[pl.ds(NRAYS, 64)])

    return k(seg2d, seg_sub, excl_flat)


def _sc_expand(segment_ids, base_ray):
    @functools.partial(
        pl.kernel,
        out_type=jax.ShapeDtypeStruct((TOTAL_N,), jnp.float32),
        mesh=_sc_mesh(),
        compiler_params=_SC_PARAMS,
        scratch_types=[
            pltpu.VMEM((NRAYS,), jnp.float32),
            pltpu.VMEM((SAMP_PT,), jnp.int32),
            pltpu.VMEM((SAMP_PT,), jnp.float32),
        ],
    )
    def k(seg_hbm, base_hbm, out_hbm, tab_r, seg_r, out_r):
        wid = _wid()
        sbase = wid * SAMP_PT
        pltpu.sync_copy(base_hbm, tab_r)
        pltpu.sync_copy(seg_hbm.at[pl.ds(sbase, SAMP_PT)], seg_r)

        def body(i, carry):
            off = i * 16
            s = seg_r[pl.ds(off, 16)]
            out_r[pl.ds(off, 16)] = plsc.load_gather(tab_r, [s])
            return carry

        lax.fori_loop(0, SAMP_PT // 16, body, 0, unroll=8)
        pltpu.sync_copy(out_r, out_hbm.at[pl.ds(sbase, SAMP_PT)])

    return k(segment_ids, base_ray)


def _sc_finalize(csum_flat, start_pad):
    @functools.partial(
        pl.kernel,
        out_type=jax.ShapeDtypeStruct((5 * NRAYS,), jnp.float32),
        mesh=_sc_mesh(),
        compiler_params=_SC_PARAMS,
        scratch_types=[
            pltpu.VMEM((264,), jnp.int32),
            pltpu.VMEM((20, 128), jnp.int32),
            pltpu.VMEM((20, 128), jnp.float32),
            pltpu.VMEM((10, 128), jnp.float32),
            pltpu.SemaphoreType.DMA,
        ],
    )
    def k(csum_hbm, start_hbm, out_hbm, st_r, idx_r, val_r, ob_r, sem):
        wid = _wid()
        rbase = wid * RAYS_PT
        pltpu.sync_copy(start_hbm.at[pl.ds(rbase, 264)], st_r)
        for j in range(2):
            for kk in range(8):
                iv = j * 128 + kk * 16 + _IOTA16()
                s = plsc.load_gather(st_r, [iv])
                e = plsc.load_gather(st_r, [iv + 1])
                ps = jnp.maximum(s - 1, 0)
                pe = jnp.maximum(e - 1, 0)
                for c in range(5):
                    idx_r.at[(c * 2) * 2 + j][pl.ds(kk * 16, 16)] = (
                        ps + c * TOTAL_N)
                    idx_r.at[(c * 2 + 1) * 2 + j][pl.ds(kk * 16, 16)] = (
                        pe + c * TOTAL_N)
        copies = [pltpu.async_copy(csum_hbm.at[idx_r.at[row]],
                                   val_r.at[row], sem)
                  for row in range(20)]
        for cp in copies:
            cp.wait()
        zero = jnp.zeros((16,), jnp.float32)
        for j in range(2):
            for kk in range(8):
                iv = j * 128 + kk * 16 + _IOTA16()
                s = plsc.load_gather(st_r, [iv])
                e = plsc.load_gather(st_r, [iv + 1])
                nonempty = e > s
                haveprev = s > 0
                for c in range(5):
                    vs = val_r.at[(c * 2) * 2 + j][pl.ds(kk * 16, 16)]
                    ve = val_r.at[(c * 2 + 1) * 2 + j][pl.ds(kk * 16, 16)]
                    res = jnp.where(
                        nonempty,
                        ve - jnp.where(haveprev, vs, zero), zero)
                    ob_r.at[c * 2 + j][pl.ds(kk * 16, 16)] = res
        for c in range(5):
            for j in range(2):
                pltpu.sync_copy(ob_r.at[c * 2 + j],
                                out_hbm.at[pl.ds(c * NRAYS + rbase + 128 * j,
                                                 128)])

    return k(csum_flat, start_pad)


def kernel(sigmas, rgbs, deltas, ts, segment_ids):
    excl_flat = _excl_cumsum(sigmas, deltas).reshape(TOTAL_N)
    seg2d = segment_ids.reshape(NSUB, WIN)
    seg_sub = seg2d[:, 0]
    start_pad, base_ray = _sc_start_base(seg2d, seg_sub, excl_flat)
    base_sample = _sc_expand(segment_ids, base_ray)
    rc, gc, bc = rgbs[:, 0], rgbs[:, 1], rgbs[:, 2]
    ws2, csum = _pass2(sigmas, deltas, ts, rc, gc, bc, excl_flat, base_sample)
    outs = _sc_finalize(csum.reshape(5 * TOTAL_N), start_pad).reshape(5, NRAYS)
    opacity = outs[0]
    depth = outs[1]
    rgb = outs[2:5].T
    return opacity, depth, rgb, ws2.reshape(TOTAL_N)


# speedup vs baseline: 78.3959x; 1.1275x over previous
import functools

import jax
import jax.numpy as jnp
from jax import lax
from jax.experimental import pallas as pl
from jax.experimental.pallas import tpu as pltpu
from jax.experimental.pallas import tpu_sc as plsc

TOTAL_N = 524288
NRAYS = 8192
ROWS = 4096
BLK_R = 512
NSTEPS = ROWS // BLK_R
LOG_EPS = -23.025850929940457
T_THRESH = 1e-4

NTILES = 32
RAYS_PT = NRAYS // NTILES
SAMP_PT = TOTAL_N // NTILES
START_PAD = NRAYS + 64
WIN = 128
NSUB = TOTAL_N // WIN

_DOT = functools.partial(jnp.dot, preferred_element_type=jnp.float32,
                         precision=jax.lax.Precision.HIGHEST)


def _tri_incl(k):
    a = jax.lax.broadcasted_iota(jnp.int32, (k, k), 0)
    b = jax.lax.broadcasted_iota(jnp.int32, (k, k), 1)
    return (a <= b).astype(jnp.float32)


def _tri_strict(k):
    a = jax.lax.broadcasted_iota(jnp.int32, (k, k), 0)
    b = jax.lax.broadcasted_iota(jnp.int32, (k, k), 1)
    return (a > b).astype(jnp.float32)


def _scan_lanes(x):
    lane = jax.lax.broadcasted_iota(jnp.int32, x.shape, 1)
    k = 1
    while k < x.shape[1]:
        x = x + jnp.where(lane >= k, pltpu.roll(x, k, 1), 0.0)
        k *= 2
    return x


def _scan_sub(x):
    row = jax.lax.broadcasted_iota(jnp.int32, x.shape, 0)
    k = 1
    while k < x.shape[0]:
        x = x + jnp.where(row >= k, pltpu.roll(x, k, 0), 0.0)
        k *= 2
    return x


def _block_cumsum(A, carry):
    S = _scan_lanes(A)
    G = _scan_sub(S)
    off = G[:, 127:128] - S[:, 127:128]
    incl = carry + off + S
    return incl, jnp.sum(A)


def _block_cumsum_multi(chans, carries):
    U = _tri_incl(128)
    Ss = [jnp.dot(A, U, preferred_element_type=jnp.float32) for A in chans]
    rt = jnp.concatenate([S[:, 127:128] for S in Ss], axis=1)
    G = _scan_sub(rt)
    off = G - rt
    incls = [carries[c] + off[:, c:c + 1] + Ss[c]
             for c in range(len(chans))]
    tots = [jnp.sum(A) for A in chans]
    return incls, tots


def _excl_kernel(sig_ref, dlt_ref, out_ref, carry_ref):
    i = pl.program_id(0)

    @pl.when(i == 0)
    def _init():
        carry_ref[0] = 0.0

    A = jnp.maximum(-(sig_ref[...] * dlt_ref[...]), LOG_EPS)
    carry = carry_ref[0]
    S = _DOT(A, _tri_incl(128))
    G = _scan_sub(S)
    off = G[:, 127:128] - S[:, 127:128]
    out_ref[...] = carry + off + (S - A)
    carry_ref[0] = carry + jnp.sum(A)


def _excl_cumsum(sigmas, deltas):
    spec = pl.BlockSpec((BLK_R, 128), lambda i: (i, 0))
    return pl.pallas_call(
        _excl_kernel,
        grid=(NSTEPS,),
        in_specs=[spec, spec],
        out_specs=spec,
        out_shape=jax.ShapeDtypeStruct((ROWS, 128), jnp.float32),
        scratch_shapes=[pltpu.SMEM((1,), jnp.float32)],
    )(sigmas.reshape(ROWS, 128), deltas.reshape(ROWS, 128))


def _pass2_kernel(sig_ref, dlt_ref, ts_ref, r_ref, g_ref, b_ref,
                  excl_ref, base_ref, ws_ref, csum_ref, carry_ref):
    i = pl.program_id(0)

    @pl.when(i == 0)
    def _init():
        for c in range(5):
            carry_ref[c] = 0.0

    sd = sig_ref[...] * dlt_ref[...]
    alpha = 1.0 - jnp.exp(-sd)
    T = jnp.exp(excl_ref[...] - base_ref[...])
    ws = jnp.where(T > T_THRESH, T * alpha, 0.0)
    ws_ref[...] = ws
    chans = (ws, ws * ts_ref[...], ws * r_ref[...], ws * g_ref[...],
             ws * b_ref[...])
    incls, tots = _block_cumsum_multi(chans, [carry_ref[c] for c in range(5)])
    for c in range(5):
        csum_ref[c] = incls[c]
        carry_ref[c] = carry_ref[c] + tots[c]


def _pass2(sigmas, deltas, ts, rc, gc, bc, excl_flat, base_sample):
    args = [x.reshape(ROWS, 128) for x in
            (sigmas, deltas, ts, rc, gc, bc, excl_flat, base_sample)]
    spec = pl.BlockSpec((BLK_R, 128), lambda i: (i, 0))
    cspec = pl.BlockSpec((5, BLK_R, 128), lambda i: (0, i, 0))
    return pl.pallas_call(
        _pass2_kernel,
        grid=(NSTEPS,),
        in_specs=[spec] * 8,
        out_specs=[spec, cspec],
        out_shape=[jax.ShapeDtypeStruct((ROWS, 128), jnp.float32),
                   jax.ShapeDtypeStruct((5, ROWS, 128), jnp.float32)],
        scratch_shapes=[pltpu.SMEM((5,), jnp.float32)],
    )(*args)


def _sc_mesh():
    return plsc.VectorSubcoreMesh(core_axis_name="c", subcore_axis_name="s")


_SC_PARAMS = pltpu.CompilerParams(needs_layout_passes=False)


def _wid():
    return lax.axis_index("c") * 16 + lax.axis_index("s")


_IOTA16 = functools.partial(lax.iota, jnp.int32, 16)


def _sc_start_base(seg2d, seg_sub, excl_flat):
    @functools.partial(
        pl.kernel,
        out_type=[jax.ShapeDtypeStruct((START_PAD,), jnp.int32),
                  jax.ShapeDtypeStruct((NRAYS,), jnp.float32)],
        mesh=_sc_mesh(),
        compiler_params=_SC_PARAMS,
        scratch_types=[
            pltpu.VMEM((NSUB,), jnp.int32),
            pltpu.VMEM((2, 128), jnp.int32),
            pltpu.VMEM((256, 128), jnp.int32),
            pltpu.VMEM((2, 128), jnp.float32),
            pltpu.VMEM((64,), jnp.int32),
            pltpu.SemaphoreType.DMA,
            pltpu.SemaphoreType.DMA,
        ],
    )
    def k(seg2d_hbm, sub_hbm, excl_hbm, start_hbm, base_hbm,
          sub_r, wi_r, rows_r, bv_r, pad_r, sem0, sem1):
        wid = _wid()
        rbase = wid * RAYS_PT
        total = jnp.full((16,), TOTAL_N, jnp.int32)
        pltpu.sync_copy(sub_hbm, sub_r)

        cs_vecs = []
        for j in range(2):
            for kk in range(8):
                r = rbase + j * 128 + kk * 16 + _IOTA16()
                lo = jnp.zeros((16,), jnp.int32)
                hi = jnp.full((16,), NSUB, jnp.int32)
                for _ in range(13):
                    mid = lax.shift_right_logical(lo + hi, 1)
                    sv = plsc.load_gather(
                        sub_r, [jnp.minimum(mid, NSUB - 1)])
                    active = lo < hi
                    pred = sv < r
                    lo = jnp.where(active, jnp.where(pred, mid + 1, lo), lo)
                    hi = jnp.where(active, jnp.where(pred, hi, mid), hi)
                cs_vecs.append(lo)
                wi_r.at[j][pl.ds(kk * 16, 16)] = jnp.maximum(lo - 1, 0)

        d0 = pltpu.async_copy(seg2d_hbm.at[wi_r.at[0]],
                              rows_r.at[pl.ds(0, 128)], sem0)
        d1 = pltpu.async_copy(seg2d_hbm.at[wi_r.at[1]],
                              rows_r.at[pl.ds(128, 128)], sem1)
        d0.wait()
        d1.wait()

        for j in range(2):
            for kk in range(8):
                v = j * 8 + kk
                cs = cs_vecs[v]
                r = rbase + j * 128 + kk * 16 + _IOTA16()
                lr = j * 128 + kk * 16 + _IOTA16()
                lo = jnp.ones((16,), jnp.int32)
                hi = jnp.full((16,), WIN, jnp.int32)
                for _ in range(7):
                    mid = lax.shift_right_logical(lo + hi, 1)
                    sv = plsc.load_gather(rows_r, [lr, mid])
                    active = lo < hi
                    pred = sv < r
                    lo = jnp.where(active, jnp.where(pred, mid + 1, lo), lo)
                    hi = jnp.where(active, jnp.where(pred, hi, mid), hi)
                w = jnp.maximum(cs - 1, 0)
                st = jnp.where(cs > 0, w * WIN + lo, 0)
                wi_r.at[j][pl.ds(kk * 16, 16)] = st

        for j in range(2):
            pltpu.sync_copy(wi_r.at[j], start_hbm.at[pl.ds(
                rbase + 128 * j, 128)])
        for j in range(2):
            for kk in range(8):
                st = wi_r.at[j][pl.ds(kk * 16, 16)]
                wi_r.at[j][pl.ds(kk * 16, 16)] = jnp.minimum(st, TOTAL_N - 1)
        d0 = pltpu.async_copy(excl_hbm.at[wi_r.at[0]], bv_r.at[0], sem0)
        d1 = pltpu.async_copy(excl_hbm.at[wi_r.at[1]], bv_r.at[1], sem1)
        d0.wait()
        d1.wait()
        for j in range(2):
            pltpu.sync_copy(bv_r.at[j], base_hbm.at[pl.ds(
                rbase + 128 * j, 128)])

        @pl.when(wid == NTILES - 1)
        def _pad():
            for kk in range(4):
                pad_r[pl.ds(kk * 16, 16)] = total
            pltpu.sync_copy(pad_r, start_hbm.at[pl.ds(NRAYS, 64)])

    return k(seg2d, seg_sub, excl_flat)


def _sc_expand(segment_ids, base_ray):
    @functools.partial(
        pl.kernel,
        out_type=jax.ShapeDtypeStruct((TOTAL_N,), jnp.float32),
        mesh=_sc_mesh(),
        compiler_params=_SC_PARAMS,
        scratch_types=[
            pltpu.VMEM((NRAYS,), jnp.float32),
            pltpu.VMEM((SAMP_PT,), jnp.int32),
            pltpu.VMEM((SAMP_PT,), jnp.float32),
        ],
    )
    def k(seg_hbm, base_hbm, out_hbm, tab_r, seg_r, out_r):
        wid = _wid()
        sbase = wid * SAMP_PT
        pltpu.sync_copy(base_hbm, tab_r)
        pltpu.sync_copy(seg_hbm.at[pl.ds(sbase, SAMP_PT)], seg_r)

        def body(i, carry):
            off = i * 16
            s = seg_r[pl.ds(off, 16)]
            out_r[pl.ds(off, 16)] = plsc.load_gather(tab_r, [s])
            return carry

        lax.fori_loop(0, SAMP_PT // 16, body, 0, unroll=16)
        pltpu.sync_copy(out_r, out_hbm.at[pl.ds(sbase, SAMP_PT)])

    return k(segment_ids, base_ray)


def _sc_finalize(csum_flat, start_pad):
    @functools.partial(
        pl.kernel,
        out_type=jax.ShapeDtypeStruct((5 * NRAYS,), jnp.float32),
        mesh=_sc_mesh(),
        compiler_params=_SC_PARAMS,
        scratch_types=[
            pltpu.VMEM((264,), jnp.int32),
            pltpu.VMEM((20, 128), jnp.int32),
            pltpu.VMEM((20, 128), jnp.float32),
            pltpu.VMEM((10, 128), jnp.float32),
            pltpu.SemaphoreType.DMA,
        ],
    )
    def k(csum_hbm, start_hbm, out_hbm, st_r, idx_r, val_r, ob_r, sem):
        wid = _wid()
        rbase = wid * RAYS_PT
        pltpu.sync_copy(start_hbm.at[pl.ds(rbase, 264)], st_r)
        for j in range(2):
            for kk in range(8):
                iv = j * 128 + kk * 16 + _IOTA16()
                s = plsc.load_gather(st_r, [iv])
                e = plsc.load_gather(st_r, [iv + 1])
                ps = jnp.maximum(s - 1, 0)
                pe = jnp.maximum(e - 1, 0)
                for c in range(5):
                    idx_r.at[(c * 2) * 2 + j][pl.ds(kk * 16, 16)] = (
                        ps + c * TOTAL_N)
                    idx_r.at[(c * 2 + 1) * 2 + j][pl.ds(kk * 16, 16)] = (
                        pe + c * TOTAL_N)
        copies = [pltpu.async_copy(csum_hbm.at[idx_r.at[row]],
                                   val_r.at[row], sem)
                  for row in range(20)]
        for cp in copies:
            cp.wait()
        zero = jnp.zeros((16,), jnp.float32)
        for j in range(2):
            for kk in range(8):
                iv = j * 128 + kk * 16 + _IOTA16()
                s = plsc.load_gather(st_r, [iv])
                e = plsc.load_gather(st_r, [iv + 1])
                nonempty = e > s
                haveprev = s > 0
                for c in range(5):
                    vs = val_r.at[(c * 2) * 2 + j][pl.ds(kk * 16, 16)]
                    ve = val_r.at[(c * 2 + 1) * 2 + j][pl.ds(kk * 16, 16)]
                    res = jnp.where(
                        nonempty,
                        ve - jnp.where(haveprev, vs, zero), zero)
                    ob_r.at[c * 2 + j][pl.ds(kk * 16, 16)] = res
        for c in range(5):
            for j in range(2):
                pltpu.sync_copy(ob_r.at[c * 2 + j],
                                out_hbm.at[pl.ds(c * NRAYS + rbase + 128 * j,
                                                 128)])

    return k(csum_flat, start_pad)


def kernel(sigmas, rgbs, deltas, ts, segment_ids):
    excl_flat = _excl_cumsum(sigmas, deltas).reshape(TOTAL_N)
    seg2d = segment_ids.reshape(NSUB, WIN)
    seg_sub = seg2d[:, 0]
    start_pad, base_ray = _sc_start_base(seg2d, seg_sub, excl_flat)
    base_sample = _sc_expand(segment_ids, base_ray)
    rc, gc, bc = rgbs[:, 0], rgbs[:, 1], rgbs[:, 2]
    ws2, csum = _pass2(sigmas, deltas, ts, rc, gc, bc, excl_flat, base_sample)
    outs = _sc_finalize(csum.reshape(5 * TOTAL_N), start_pad).reshape(5, NRAYS)
    opacity = outs[0]
    depth = outs[1]
    rgb = outs[2:5].T
    return opacity, depth, rgb, ws2.reshape(TOTAL_N)


# speedup vs baseline: 85.7223x; 1.0935x over previous
import functools

import jax
import jax.numpy as jnp
from jax import lax
from jax.experimental import pallas as pl
from jax.experimental.pallas import tpu as pltpu
from jax.experimental.pallas import tpu_sc as plsc

TOTAL_N = 524288
NRAYS = 8192
ROWS = 4096
BLK_R = 1024
NSTEPS = ROWS // BLK_R
LOG_EPS = -23.025850929940457
T_THRESH = 1e-4

NTILES = 32
RAYS_PT = NRAYS // NTILES
SAMP_PT = TOTAL_N // NTILES
START_PAD = NRAYS + 64
WIN = 128
NSUB = TOTAL_N // WIN

_DOT = functools.partial(jnp.dot, preferred_element_type=jnp.float32,
                         precision=jax.lax.Precision.HIGHEST)


def _tri_incl(k):
    a = jax.lax.broadcasted_iota(jnp.int32, (k, k), 0)
    b = jax.lax.broadcasted_iota(jnp.int32, (k, k), 1)
    return (a <= b).astype(jnp.float32)


def _tri_strict(k):
    a = jax.lax.broadcasted_iota(jnp.int32, (k, k), 0)
    b = jax.lax.broadcasted_iota(jnp.int32, (k, k), 1)
    return (a > b).astype(jnp.float32)


def _scan_lanes(x):
    lane = jax.lax.broadcasted_iota(jnp.int32, x.shape, 1)
    k = 1
    while k < x.shape[1]:
        x = x + jnp.where(lane >= k, pltpu.roll(x, k, 1), 0.0)
        k *= 2
    return x


def _scan_sub(x):
    row = jax.lax.broadcasted_iota(jnp.int32, x.shape, 0)
    k = 1
    while k < x.shape[0]:
        x = x + jnp.where(row >= k, pltpu.roll(x, k, 0), 0.0)
        k *= 2
    return x


def _block_cumsum(A, carry):
    S = _scan_lanes(A)
    G = _scan_sub(S)
    off = G[:, 127:128] - S[:, 127:128]
    incl = carry + off + S
    return incl, jnp.sum(A)


def _block_cumsum_multi(chans, carries):
    U = _tri_incl(128)
    Ss = [jnp.dot(A, U, preferred_element_type=jnp.float32) for A in chans]
    rt = jnp.concatenate([S[:, 127:128] for S in Ss], axis=1)
    G = _scan_sub(rt)
    off = G - rt
    incls = [carries[c] + off[:, c:c + 1] + Ss[c]
             for c in range(len(chans))]
    tots = [jnp.sum(A) for A in chans]
    return incls, tots


def _excl_kernel(sig_ref, dlt_ref, out_ref, carry_ref):
    i = pl.program_id(0)

    @pl.when(i == 0)
    def _init():
        carry_ref[0] = 0.0

    A = jnp.maximum(-(sig_ref[...] * dlt_ref[...]), LOG_EPS)
    carry = carry_ref[0]
    S = _DOT(A, _tri_incl(128))
    G = _scan_sub(S)
    off = G[:, 127:128] - S[:, 127:128]
    out_ref[...] = carry + off + (S - A)
    carry_ref[0] = carry + jnp.sum(A)


def _excl_cumsum(sigmas, deltas):
    spec = pl.BlockSpec((BLK_R, 128), lambda i: (i, 0))
    return pl.pallas_call(
        _excl_kernel,
        grid=(NSTEPS,),
        in_specs=[spec, spec],
        out_specs=spec,
        out_shape=jax.ShapeDtypeStruct((ROWS, 128), jnp.float32),
        scratch_shapes=[pltpu.SMEM((1,), jnp.float32)],
    )(sigmas.reshape(ROWS, 128), deltas.reshape(ROWS, 128))


def _pass2_kernel(sig_ref, dlt_ref, ts_ref, r_ref, g_ref, b_ref,
                  excl_ref, base_ref, ws_ref, csum_ref, carry_ref):
    i = pl.program_id(0)

    @pl.when(i == 0)
    def _init():
        for c in range(5):
            carry_ref[c] = 0.0

    sd = sig_ref[...] * dlt_ref[...]
    alpha = 1.0 - jnp.exp(-sd)
    T = jnp.exp(excl_ref[...] - base_ref[...])
    ws = jnp.where(T > T_THRESH, T * alpha, 0.0)
    ws_ref[...] = ws
    chans = (ws, ws * ts_ref[...], ws * r_ref[...], ws * g_ref[...],
             ws * b_ref[...])
    incls, tots = _block_cumsum_multi(chans, [carry_ref[c] for c in range(5)])
    for c in range(5):
        csum_ref[c] = incls[c]
        carry_ref[c] = carry_ref[c] + tots[c]


def _pass2(sigmas, deltas, ts, rc, gc, bc, excl_flat, base_sample):
    args = [x.reshape(ROWS, 128) for x in
            (sigmas, deltas, ts, rc, gc, bc, excl_flat, base_sample)]
    spec = pl.BlockSpec((BLK_R, 128), lambda i: (i, 0))
    cspec = pl.BlockSpec((5, BLK_R, 128), lambda i: (0, i, 0))
    return pl.pallas_call(
        _pass2_kernel,
        grid=(NSTEPS,),
        in_specs=[spec] * 8,
        out_specs=[spec, cspec],
        out_shape=[jax.ShapeDtypeStruct((ROWS, 128), jnp.float32),
                   jax.ShapeDtypeStruct((5, ROWS, 128), jnp.float32)],
        scratch_shapes=[pltpu.SMEM((5,), jnp.float32)],
    )(*args)


def _sc_mesh():
    return plsc.VectorSubcoreMesh(core_axis_name="c", subcore_axis_name="s")


_SC_PARAMS = pltpu.CompilerParams(needs_layout_passes=False)


def _wid():
    return lax.axis_index("c") * 16 + lax.axis_index("s")


_IOTA16 = functools.partial(lax.iota, jnp.int32, 16)


def _sc_start_base(seg2d, seg_sub, excl_flat):
    @functools.partial(
        pl.kernel,
        out_type=[jax.ShapeDtypeStruct((START_PAD,), jnp.int32),
                  jax.ShapeDtypeStruct((NRAYS,), jnp.float32)],
        mesh=_sc_mesh(),
        compiler_params=_SC_PARAMS,
        scratch_types=[
            pltpu.VMEM((NSUB,), jnp.int32),
            pltpu.VMEM((2, 128), jnp.int32),
            pltpu.VMEM((256, 128), jnp.int32),
            pltpu.VMEM((2, 128), jnp.float32),
            pltpu.VMEM((64,), jnp.int32),
            pltpu.SemaphoreType.DMA,
            pltpu.SemaphoreType.DMA,
        ],
    )
    def k(seg2d_hbm, sub_hbm, excl_hbm, start_hbm, base_hbm,
          sub_r, wi_r, rows_r, bv_r, pad_r, sem0, sem1):
        wid = _wid()
        rbase = wid * RAYS_PT
        total = jnp.full((16,), TOTAL_N, jnp.int32)
        pltpu.sync_copy(sub_hbm, sub_r)

        cs_vecs = []
        for j in range(2):
            for kk in range(8):
                r = rbase + j * 128 + kk * 16 + _IOTA16()
                lo = jnp.zeros((16,), jnp.int32)
                hi = jnp.full((16,), NSUB, jnp.int32)
                for _ in range(13):
                    mid = lax.shift_right_logical(lo + hi, 1)
                    sv = plsc.load_gather(
                        sub_r, [jnp.minimum(mid, NSUB - 1)])
                    active = lo < hi
                    pred = sv < r
                    lo = jnp.where(active, jnp.where(pred, mid + 1, lo), lo)
                    hi = jnp.where(active, jnp.where(pred, hi, mid), hi)
                cs_vecs.append(lo)
                wi_r.at[j][pl.ds(kk * 16, 16)] = jnp.maximum(lo - 1, 0)

        d0 = pltpu.async_copy(seg2d_hbm.at[wi_r.at[0]],
                              rows_r.at[pl.ds(0, 128)], sem0)
        d1 = pltpu.async_copy(seg2d_hbm.at[wi_r.at[1]],
                              rows_r.at[pl.ds(128, 128)], sem1)
        d0.wait()
        d1.wait()

        for j in range(2):
            for kk in range(8):
                v = j * 8 + kk
                cs = cs_vecs[v]
                r = rbase + j * 128 + kk * 16 + _IOTA16()
                lr = j * 128 + kk * 16 + _IOTA16()
                lo = jnp.ones((16,), jnp.int32)
                hi = jnp.full((16,), WIN, jnp.int32)
                for _ in range(7):
                    mid = lax.shift_right_logical(lo + hi, 1)
                    sv = plsc.load_gather(rows_r, [lr, mid])
                    active = lo < hi
                    pred = sv < r
                    lo = jnp.where(active, jnp.where(pred, mid + 1, lo), lo)
                    hi = jnp.where(active, jnp.where(pred, hi, mid), hi)
                w = jnp.maximum(cs - 1, 0)
                st = jnp.where(cs > 0, w * WIN + lo, 0)
                wi_r.at[j][pl.ds(kk * 16, 16)] = st

        for j in range(2):
            pltpu.sync_copy(wi_r.at[j], start_hbm.at[pl.ds(
                rbase + 128 * j, 128)])
        for j in range(2):
            for kk in range(8):
                st = wi_r.at[j][pl.ds(kk * 16, 16)]
                wi_r.at[j][pl.ds(kk * 16, 16)] = jnp.minimum(st, TOTAL_N - 1)
        d0 = pltpu.async_copy(excl_hbm.at[wi_r.at[0]], bv_r.at[0], sem0)
        d1 = pltpu.async_copy(excl_hbm.at[wi_r.at[1]], bv_r.at[1], sem1)
        d0.wait()
        d1.wait()
        for j in range(2):
            pltpu.sync_copy(bv_r.at[j], base_hbm.at[pl.ds(
                rbase + 128 * j, 128)])

        @pl.when(wid == NTILES - 1)
        def _pad():
            for kk in range(4):
                pad_r[pl.ds(kk * 16, 16)] = total
            pltpu.sync_copy(pad_r, start_hbm.at[pl.ds(NRAYS, 64)])

    return k(seg2d, seg_sub, excl_flat)


def _sc_expand(segment_ids, base_ray):
    @functools.partial(
        pl.kernel,
        out_type=jax.ShapeDtypeStruct((TOTAL_N,), jnp.float32),
        mesh=_sc_mesh(),
        compiler_params=_SC_PARAMS,
        scratch_types=[
            pltpu.VMEM((NRAYS,), jnp.float32),
            pltpu.VMEM((SAMP_PT,), jnp.int32),
            pltpu.VMEM((SAMP_PT,), jnp.float32),
        ],
    )
    def k(seg_hbm, base_hbm, out_hbm, tab_r, seg_r, out_r):
        wid = _wid()
        sbase = wid * SAMP_PT
        pltpu.sync_copy(base_hbm, tab_r)
        pltpu.sync_copy(seg_hbm.at[pl.ds(sbase, SAMP_PT)], seg_r)

        @plsc.parallel_loop(0, SAMP_PT, step=16, unroll=16)
        def _body(off):
            s = seg_r[pl.ds(off, 16)]
            out_r[pl.ds(off, 16)] = plsc.load_gather(tab_r, [s])
        pltpu.sync_copy(out_r, out_hbm.at[pl.ds(sbase, SAMP_PT)])

    return k(segment_ids, base_ray)


def _sc_finalize(csum_flat, start_pad):
    @functools.partial(
        pl.kernel,
        out_type=jax.ShapeDtypeStruct((5 * NRAYS,), jnp.float32),
        mesh=_sc_mesh(),
        compiler_params=_SC_PARAMS,
        scratch_types=[
            pltpu.VMEM((264,), jnp.int32),
            pltpu.VMEM((20, 128), jnp.int32),
            pltpu.VMEM((20, 128), jnp.float32),
            pltpu.VMEM((10, 128), jnp.float32),
            pltpu.SemaphoreType.DMA,
        ],
    )
    def k(csum_hbm, start_hbm, out_hbm, st_r, idx_r, val_r, ob_r, sem):
        wid = _wid()
        rbase = wid * RAYS_PT
        pltpu.sync_copy(start_hbm.at[pl.ds(rbase, 264)], st_r)
        for j in range(2):
            for kk in range(8):
                iv = j * 128 + kk * 16 + _IOTA16()
                s = plsc.load_gather(st_r, [iv])
                e = plsc.load_gather(st_r, [iv + 1])
                ps = jnp.maximum(s - 1, 0)
                pe = jnp.maximum(e - 1, 0)
                for c in range(5):
                    idx_r.at[(c * 2) * 2 + j][pl.ds(kk * 16, 16)] = (
                        ps + c * TOTAL_N)
                    idx_r.at[(c * 2 + 1) * 2 + j][pl.ds(kk * 16, 16)] = (
                        pe + c * TOTAL_N)
        copies = [pltpu.async_copy(csum_hbm.at[idx_r.at[row]],
                                   val_r.at[row], sem)
                  for row in range(20)]
        for cp in copies:
            cp.wait()
        zero = jnp.zeros((16,), jnp.float32)
        for j in range(2):
            for kk in range(8):
                iv = j * 128 + kk * 16 + _IOTA16()
                s = plsc.load_gather(st_r, [iv])
                e = plsc.load_gather(st_r, [iv + 1])
                nonempty = e > s
                haveprev = s > 0
                for c in range(5):
                    vs = val_r.at[(c * 2) * 2 + j][pl.ds(kk * 16, 16)]
                    ve = val_r.at[(c * 2 + 1) * 2 + j][pl.ds(kk * 16, 16)]
                    res = jnp.where(
                        nonempty,
                        ve - jnp.where(haveprev, vs, zero), zero)
                    ob_r.at[c * 2 + j][pl.ds(kk * 16, 16)] = res
        for c in range(5):
            for j in range(2):
                pltpu.sync_copy(ob_r.at[c * 2 + j],
                                out_hbm.at[pl.ds(c * NRAYS + rbase + 128 * j,
                                                 128)])

    return k(csum_flat, start_pad)


def kernel(sigmas, rgbs, deltas, ts, segment_ids):
    excl_flat = _excl_cumsum(sigmas, deltas).reshape(TOTAL_N)
    seg2d = segment_ids.reshape(NSUB, WIN)
    seg_sub = seg2d[:, 0]
    start_pad, base_ray = _sc_start_base(seg2d, seg_sub, excl_flat)
    base_sample = _sc_expand(segment_ids, base_ray)
    rc, gc, bc = rgbs[:, 0], rgbs[:, 1], rgbs[:, 2]
    ws2, csum = _pass2(sigmas, deltas, ts, rc, gc, bc, excl_flat, base_sample)
    outs = _sc_finalize(csum.reshape(5 * TOTAL_N), start_pad).reshape(5, NRAYS)
    opacity = outs[0]
    depth = outs[1]
    rgb = outs[2:5].T
    return opacity, depth, rgb, ws2.reshape(TOTAL_N)
